# baseline (speedup 1.0000x reference)
import functools

import jax
import jax.numpy as jnp
import numpy as np
from jax import lax
from jax.experimental import pallas as pl
from jax.experimental.pallas import tpu as pltpu
from jax.experimental.pallas import tpu_sc as plsc

B = 16384
N_CONT = 13
N_CAT = 26
V = 100001
D = 16
EPS = 1e-5

V_PAD = 100352
PLANE = D * V_PAD

NC = 2
NS = 16
NW = NC * NS
ROWS_PER_W = B // NW
BCHUNK = 512
N_BCHUNK = ROWS_PER_W // BCHUNK

CB = 2048


HALF = N_CAT // 2


def _tc_repack(tab_t_half):
    def body(t_ref, o_ref):
        o_ref[...] = t_ref[0].reshape(PLANE)

    return pl.pallas_call(
        body,
        grid=(HALF,),
        in_specs=[pl.BlockSpec((1, D, V_PAD), lambda i: (i, 0, 0))],
        out_specs=pl.BlockSpec((PLANE,), lambda i: (i,)),
        out_shape=jax.ShapeDtypeStruct((HALF * PLANE,), jnp.float32),
        compiler_params=pltpu.CompilerParams(
            dimension_semantics=("arbitrary",),
        ),
    )(tab_t_half)


def _sc_gather(cat_t_half, tab_lin_half):
    mesh = plsc.VectorSubcoreMesh(
        core_axis_name="c", subcore_axis_name="s", num_cores=NC, num_subcores=NS
    )

    @functools.partial(
        pl.kernel,
        out_type=jax.ShapeDtypeStruct((HALF, D, B), jnp.float32),
        name="sc_embedding_gather",
        mesh=mesh,
        scratch_types=[
            pltpu.VMEM((2, BCHUNK), jnp.int32),
            pltpu.VMEM((2, D, BCHUNK), jnp.float32),
            pltpu.SemaphoreType.DMA,
            pltpu.SemaphoreType.DMA,
            pltpu.SemaphoreType.DMA,
            pltpu.SemaphoreType.DMA,
        ],
        compiler_params=pltpu.CompilerParams(use_tc_tiling_on_sc=False),
    )
    def body(cat_hbm, tab_hbm, out_hbm, idx_v, val_v, sem_i, sem_g0, sem_g1, sem_o):
        wid = lax.axis_index("s") * NC + lax.axis_index("c")
        base = wid * ROWS_PER_W

        def fire(t, par):
            def one(d, carry):
                off = pl.multiple_of((t * D + d) * V_PAD, 128)
                sem = [sem_g0, sem_g1][par]
                pltpu.async_copy(
                    tab_hbm.at[pl.ds(off, V_PAD)].at[idx_v.at[par]],
                    val_v.at[par, d],
                    sem,
                )
                return carry

            lax.fori_loop(0, D, one, 0)

        def drain_val(par):
            sem = [sem_g0, sem_g1][par]
            pltpu.make_async_copy(
                out_hbm.at[0, :, pl.ds(0, BCHUNK)], val_v.at[par], sem
            ).wait()

        def drain_out():
            pltpu.make_async_copy(
                val_v.at[0], out_hbm.at[0, :, pl.ds(0, BCHUNK)], sem_o
            ).wait()

        pltpu.sync_copy(cat_hbm.at[0, pl.ds(base, BCHUNK)], idx_v.at[0])
        fire(0, 0)

        def step(t, carry):
            par = lax.rem(t, 2)
            nxt = 1 - par

            @pl.when(t + 1 < HALF)
            def _():
                pltpu.async_copy(
                    cat_hbm.at[t + 1, pl.ds(base, BCHUNK)], idx_v.at[nxt], sem_i
                ).wait()

                @pl.when(t >= 1)
                def _():
                    drain_out()

                @pl.when(nxt == 0)
                def _():
                    fire(t + 1, 0)

                @pl.when(nxt == 1)
                def _():
                    fire(t + 1, 1)

            @pl.when(par == 0)
            def _():
                drain_val(0)
                pltpu.async_copy(
                    val_v.at[0], out_hbm.at[t, :, pl.ds(base, BCHUNK)], sem_o
                )

            @pl.when(par == 1)
            def _():
                drain_val(1)
                pltpu.async_copy(
                    val_v.at[1], out_hbm.at[t, :, pl.ds(base, BCHUNK)], sem_o
                )

            return carry

        lax.fori_loop(0, HALF, step, 0)
        drain_out()
        drain_out()

    return body(cat_t_half, tab_lin_half)


def _tc_cont(x_t, gamma, beta, w_t, b_t, xcat_a, xcat_b):
    def body(x_ref, xc_ref, g_ref, be_ref, w_ref, bb_ref, ca_ref, cb_ref, o_ref):
        xv = x_ref[...]
        mean = jnp.mean(xv, axis=1, keepdims=True)
        var = jnp.mean(xv * xv, axis=1, keepdims=True) - mean * mean
        inv = lax.rsqrt(var + EPS) * g_ref[...]
        xc = (xc_ref[...] - mean) * inv + be_ref[...]
        cont = w_ref[...] * xc[:, None, :] + bb_ref[...]
        o_ref[...] = jnp.concatenate(
            [cont, ca_ref[...], cb_ref[...]], axis=0
        )

    grid = B // CB
    return pl.pallas_call(
        body,
        grid=(grid,),
        in_specs=[
            pl.BlockSpec((N_CONT, B), lambda i: (0, 0)),
            pl.BlockSpec((N_CONT, CB), lambda i: (0, i)),
            pl.BlockSpec((N_CONT, 1), lambda i: (0, 0)),
            pl.BlockSpec((N_CONT, 1), lambda i: (0, 0)),
            pl.BlockSpec((N_CONT, D, 1), lambda i: (0, 0, 0)),
            pl.BlockSpec((N_CONT, D, 1), lambda i: (0, 0, 0)),
            pl.BlockSpec((HALF, D, CB), lambda i: (0, 0, i)),
            pl.BlockSpec((HALF, D, CB), lambda i: (0, 0, i)),
        ],
        out_specs=pl.BlockSpec((N_CONT + N_CAT, D, CB), lambda i: (0, 0, i)),
        out_shape=jax.ShapeDtypeStruct((N_CONT + N_CAT, D, B), jnp.float32),
        compiler_params=pltpu.CompilerParams(
            dimension_semantics=("arbitrary",),
        ),
    )(x_t, x_t, gamma, beta, w_t, b_t, xcat_a, xcat_b)


def kernel(x, categorical, cont_embed_weight, cont_embed_bias, bn_gamma, bn_beta, cat_tables):
    tab_t = cat_tables.transpose(0, 2, 1)
    cat_t = categorical.T
    x_t = x.T
    gamma = bn_gamma.reshape(N_CONT, 1)
    beta = bn_beta.reshape(N_CONT, 1)
    w_t = cont_embed_weight.reshape(N_CONT, D, 1)
    b_t = cont_embed_bias.reshape(N_CONT, D, 1)

    tab_lin_a = _tc_repack(tab_t[:HALF])
    xcat_a = _sc_gather(cat_t[:HALF], tab_lin_a)
    tab_lin_b = _tc_repack(tab_t[HALF:])
    xcat_b = _sc_gather(cat_t[HALF:], tab_lin_b)

    out_t = _tc_cont(x_t, gamma, beta, w_t, b_t, xcat_a, xcat_b)
    return out_t.transpose(2, 0, 1)


# speedup vs baseline: 1.2097x; 1.2097x over previous
import functools

import jax
import jax.numpy as jnp
import numpy as np
from jax import lax
from jax.experimental import pallas as pl
from jax.experimental.pallas import tpu as pltpu
from jax.experimental.pallas import tpu_sc as plsc

B = 16384
N_CONT = 13
N_CAT = 26
V = 100001
D = 16
EPS = 1e-5

V_PAD = 100352
PLANE = D * V_PAD

NC = 2
NS = 16
NW = NC * NS
ROWS_PER_W = B // NW
BCHUNK = 512
N_BCHUNK = ROWS_PER_W // BCHUNK

CB = 2048


def _tc_repack(tab_t):
    def body(t_ref, o_ref):
        o_ref[...] = t_ref[0].reshape(PLANE)

    return pl.pallas_call(
        body,
        grid=(N_CAT,),
        in_specs=[pl.BlockSpec((1, D, V_PAD), lambda i: (i, 0, 0))],
        out_specs=pl.BlockSpec((PLANE,), lambda i: (i,)),
        out_shape=jax.ShapeDtypeStruct((N_CAT * PLANE,), jnp.float32),
        compiler_params=pltpu.CompilerParams(
            dimension_semantics=("arbitrary",),
        ),
    )(tab_t)


def _sc_gather(cat_t, tab_lin):
    mesh = plsc.VectorSubcoreMesh(
        core_axis_name="c", subcore_axis_name="s", num_cores=NC, num_subcores=NS
    )

    @functools.partial(
        pl.kernel,
        out_type=jax.ShapeDtypeStruct((N_CONT + N_CAT, D, B), jnp.float32),
        name="sc_embedding_gather",
        mesh=mesh,
        scratch_types=[
            pltpu.VMEM((2, BCHUNK), jnp.int32),
            pltpu.VMEM((2, D, BCHUNK), jnp.float32),
            pltpu.SemaphoreType.DMA,
            pltpu.SemaphoreType.DMA,
            pltpu.SemaphoreType.DMA,
            pltpu.SemaphoreType.DMA,
        ],
        compiler_params=pltpu.CompilerParams(use_tc_tiling_on_sc=False),
    )
    def body(cat_hbm, tab_hbm, out_hbm, idx_v, val_v, sem_i, sem_g0, sem_g1, sem_o):
        wid = lax.axis_index("s") * NC + lax.axis_index("c")
        base = wid * ROWS_PER_W

        def fire(t, par):
            def one(d, carry):
                off = pl.multiple_of((t * D + d) * V_PAD, 128)
                sem = [sem_g0, sem_g1][par]
                pltpu.async_copy(
                    tab_hbm.at[pl.ds(off, V_PAD)].at[idx_v.at[par]],
                    val_v.at[par, d],
                    sem,
                )
                return carry

            lax.fori_loop(0, D, one, 0)

        def drain_val(par):
            sem = [sem_g0, sem_g1][par]
            pltpu.make_async_copy(
                out_hbm.at[0, :, pl.ds(0, BCHUNK)], val_v.at[par], sem
            ).wait()

        def drain_out():
            pltpu.make_async_copy(
                val_v.at[0], out_hbm.at[0, :, pl.ds(0, BCHUNK)], sem_o
            ).wait()

        pltpu.sync_copy(cat_hbm.at[0, pl.ds(base, BCHUNK)], idx_v.at[0])
        fire(0, 0)

        def step(t, carry):
            par = lax.rem(t, 2)
            nxt = 1 - par

            @pl.when(t + 1 < N_CAT)
            def _():
                pltpu.async_copy(
                    cat_hbm.at[t + 1, pl.ds(base, BCHUNK)], idx_v.at[nxt], sem_i
                ).wait()

                @pl.when(t >= 1)
                def _():
                    drain_out()

                @pl.when(nxt == 0)
                def _():
                    fire(t + 1, 0)

                @pl.when(nxt == 1)
                def _():
                    fire(t + 1, 1)

            @pl.when(par == 0)
            def _():
                drain_val(0)
                pltpu.async_copy(
                    val_v.at[0], out_hbm.at[N_CONT + t, :, pl.ds(base, BCHUNK)],
                    sem_o,
                )

            @pl.when(par == 1)
            def _():
                drain_val(1)
                pltpu.async_copy(
                    val_v.at[1], out_hbm.at[N_CONT + t, :, pl.ds(base, BCHUNK)],
                    sem_o,
                )

            return carry

        lax.fori_loop(0, N_CAT, step, 0)
        drain_out()
        drain_out()

    return body(cat_t, tab_lin)


def _tc_cont(x_t, gamma, beta, w_t, b_t, scout):
    def body(x_ref, xc_ref, g_ref, be_ref, w_ref, bb_ref, sc_ref, o_ref):
        xv = x_ref[...]
        mean = jnp.mean(xv, axis=1, keepdims=True)
        var = jnp.mean(xv * xv, axis=1, keepdims=True) - mean * mean
        inv = lax.rsqrt(var + EPS) * g_ref[...]
        xc = (xc_ref[...] - mean) * inv + be_ref[...]
        o_ref[...] = (
            w_ref[...] * xc[:, None, :] + bb_ref[...]
        )

    grid = B // CB
    return pl.pallas_call(
        body,
        grid=(grid,),
        in_specs=[
            pl.BlockSpec((N_CONT, B), lambda i: (0, 0)),
            pl.BlockSpec((N_CONT, CB), lambda i: (0, i)),
            pl.BlockSpec((N_CONT, 1), lambda i: (0, 0)),
            pl.BlockSpec((N_CONT, 1), lambda i: (0, 0)),
            pl.BlockSpec((N_CONT, D, 1), lambda i: (0, 0, 0)),
            pl.BlockSpec((N_CONT, D, 1), lambda i: (0, 0, 0)),
            pl.BlockSpec(memory_space=pl.ANY),
        ],
        out_specs=pl.BlockSpec((N_CONT, D, CB), lambda i: (0, 0, i)),
        out_shape=jax.ShapeDtypeStruct((N_CONT + N_CAT, D, B), jnp.float32),
        input_output_aliases={6: 0},
        compiler_params=pltpu.CompilerParams(
            dimension_semantics=("arbitrary",),
        ),
    )(x_t, x_t, gamma, beta, w_t, b_t, scout)


def kernel(x, categorical, cont_embed_weight, cont_embed_bias, bn_gamma, bn_beta, cat_tables):
    tab_t = cat_tables.transpose(0, 2, 1)
    cat_t = categorical.T
    x_t = x.T
    gamma = bn_gamma.reshape(N_CONT, 1)
    beta = bn_beta.reshape(N_CONT, 1)
    w_t = cont_embed_weight.reshape(N_CONT, D, 1)
    b_t = cont_embed_bias.reshape(N_CONT, D, 1)

    tab_lin = _tc_repack(tab_t)

    scout = _sc_gather(cat_t, tab_lin)

    out_t = _tc_cont(x_t, gamma, beta, w_t, b_t, scout)
    return out_t.transpose(2, 0, 1)
